# baseline (speedup 1.0000x reference)
import functools

import jax
import jax.numpy as jnp
from jax import lax
from jax.experimental import pallas as pl
from jax.experimental.pallas import tpu as pltpu
from jax.experimental.pallas import tpu_sc as plsc

_NC = 2
_NS = 16
_NW = _NC * _NS

_TM = 256


def _router_body(x_ref, wr_ref, br_ref, w_ref, i_ref):
    xb = x_ref[...]
    logits = lax.dot_general(
        xb, wr_ref[...], (((1,), (1,)), ((), ())),
        preferred_element_type=jnp.float32,
    ) + br_ref[...]
    e_iota = lax.broadcasted_iota(jnp.int32, logits.shape, 1)
    m1 = jnp.max(logits, axis=1, keepdims=True)
    i1 = jnp.argmax(logits, axis=1).astype(jnp.int32)
    oh1 = e_iota == i1[:, None]
    l2 = jnp.where(oh1, -jnp.inf, logits)
    m2 = jnp.max(l2, axis=1, keepdims=True)
    i2 = jnp.argmax(l2, axis=1).astype(jnp.int32)
    b = jnp.exp(m2 - m1)
    w1 = 1.0 / (1.0 + b)
    w2 = 1.0 - w1
    w_ref[...] = jnp.where(e_iota == 0, w1, jnp.where(e_iota == 1, w2, 0.0))
    i_ref[...] = jnp.where(e_iota == 0, i1[:, None],
                           jnp.where(e_iota == 1, i2[:, None], 0))


def _router(x_flat, Wr, br):
    BS, D = x_flat.shape
    E = Wr.shape[0]
    TMA = 512
    grid = (BS // TMA,)
    return pl.pallas_call(
        _router_body,
        grid=grid,
        in_specs=[
            pl.BlockSpec((TMA, D), lambda i: (i, 0)),
            pl.BlockSpec((E, D), lambda i: (0, 0)),
            pl.BlockSpec((1, E), lambda i: (0, 0)),
        ],
        out_specs=[
            pl.BlockSpec((TMA, E), lambda i: (i, 0)),
            pl.BlockSpec((TMA, E), lambda i: (i, 0)),
        ],
        out_shape=[
            jax.ShapeDtypeStruct((BS, E), jnp.float32),
            jax.ShapeDtypeStruct((BS, E), jnp.int32),
        ],
    )(x_flat, Wr, br.reshape(1, E))


def _make_sc_gather(V, D, N, CH, dtype):
    assert N % _NW == 0
    rows_pw = N // _NW
    assert rows_pw % CH == 0 and rows_pw % 8 == 0
    n_ch = rows_pw // CH
    assert n_ch % 2 == 0 and n_ch >= 4
    mesh = plsc.VectorSubcoreMesh(
        core_axis_name="c", subcore_axis_name="s",
        num_cores=_NC, num_subcores=_NS,
    )

    @functools.partial(
        pl.kernel,
        out_type=jax.ShapeDtypeStruct((N, D), dtype),
        mesh=mesh,
        scratch_types=[
            pltpu.VMEM((rows_pw,), jnp.int32),
            pltpu.VMEM((2, CH, D), dtype),
            pltpu.SemaphoreType.DMA,
            pltpu.SemaphoreType.DMA,
            pltpu.SemaphoreType.DMA,
            pltpu.SemaphoreType.DMA,
        ],
    )
    def gather(table_hbm, idx_hbm, out_hbm, idx_v, rows_v, g0, g1, w0, w1):
        wid = lax.axis_index("s") * _NC + lax.axis_index("c")
        base = wid * rows_pw
        g_sems = (g0, g1)
        w_sems = (w0, w1)
        pltpu.sync_copy(idx_hbm.at[pl.ds(base, rows_pw)], idx_v)

        def fire_gather(j, b):
            pltpu.async_copy(
                table_hbm.at[idx_v.at[pl.ds(j * CH, CH)]], rows_v.at[b],
                g_sems[b])

        fire_gather(0, 0)
        fire_gather(1, 1)

        @pl.loop(0, n_ch, step=2)
        def _pair(i):
            for b in range(2):
                j = i + b
                pltpu.make_async_copy(
                    table_hbm.at[idx_v.at[pl.ds(0, CH)]], rows_v.at[b],
                    g_sems[b]).wait()
                pltpu.async_copy(
                    rows_v.at[b], out_hbm.at[pl.ds(base + j * CH, CH)],
                    w_sems[b])

                @pl.when(j + 2 < n_ch)
                def _refill():
                    pltpu.make_async_copy(
                        rows_v.at[b], out_hbm.at[pl.ds(base, CH)],
                        w_sems[b]).wait()
                    fire_gather(j + 2, b)

        for b in range(2):
            pltpu.make_async_copy(
                rows_v.at[b], out_hbm.at[pl.ds(base, CH)], w_sems[b]).wait()

    return gather


def _expert_body(gid_ref, xs_ref, we_ref, be_ref, ws_ref, out_ref):
    g = gid_ref[pl.program_id(0)]
    xb = xs_ref[...]
    wg = we_ref[g]
    z = lax.dot_general(
        xb, wg, (((1,), (1,)), ((), ())),
        preferred_element_type=jnp.float32,
    ) + be_ref[g][None, :]
    h = z * (1.0 / (1.0 + jnp.exp(-z)))
    w = ws_ref[0, 0][:, None]
    out_ref[...] = (h * w).astype(jnp.bfloat16)


def _expert_mm(xs, We, be, w_tiles, gids):
    P, D = xs.shape
    E = We.shape[0]
    ntiles = P // _TM
    return pl.pallas_call(
        _expert_body,
        grid=(ntiles,),
        in_specs=[
            pl.BlockSpec(memory_space=pltpu.SMEM),
            pl.BlockSpec((_TM, D), lambda i: (i, 0)),
            pl.BlockSpec((E, D, D), lambda i: (0, 0, 0)),
            pl.BlockSpec((E, D), lambda i: (0, 0)),
            pl.BlockSpec((1, 1, _TM), lambda i: (i, 0, 0)),
        ],
        out_specs=pl.BlockSpec((_TM, D), lambda i: (i, 0)),
        out_shape=jax.ShapeDtypeStruct((P, D), jnp.bfloat16),
        compiler_params=pltpu.CompilerParams(
            dimension_semantics=("arbitrary",),
        ),
    )(gids, xs, We, be, w_tiles)


def _out_body(ga_ref, gb_ref, x_ref, wo_ref, bo_ref, g_ref, o_ref):
    c = (ga_ref[...].astype(jnp.float32)
         + gb_ref[...].astype(jnp.float32))
    z = lax.dot_general(
        c, wo_ref[...], (((1,), (1,)), ((), ())),
        preferred_element_type=jnp.float32,
    ) + bo_ref[...]
    y = x_ref[...] + z
    ms = jnp.mean(y * y, axis=1, keepdims=True)
    o_ref[...] = g_ref[...] * (y * lax.rsqrt(ms + 1e-6))


def _out_proj(gab, x_flat, Wo, bo, g):
    BS, D = x_flat.shape
    TMD = 256
    nb = BS // TMD
    return pl.pallas_call(
        _out_body,
        grid=(nb,),
        in_specs=[
            pl.BlockSpec((TMD, D), lambda i: (i, 0)),
            pl.BlockSpec((TMD, D), lambda i, nb=nb: (i + nb, 0)),
            pl.BlockSpec((TMD, D), lambda i: (i, 0)),
            pl.BlockSpec((D, D), lambda i: (0, 0)),
            pl.BlockSpec((1, D), lambda i: (0, 0)),
            pl.BlockSpec((1, D), lambda i: (0, 0)),
        ],
        out_specs=pl.BlockSpec((TMD, D), lambda i: (i, 0)),
        out_shape=jax.ShapeDtypeStruct((BS, D), jnp.float32),
    )(gab, gab, x_flat, Wo, bo.reshape(1, D), g.reshape(1, D))


def kernel(x, Wr, br, We, be, Wo, bo, g):
    B, S, D = x.shape
    E = Wr.shape[0]
    K = 2
    BS = B * S
    A = BS * K
    P = A + E * _TM

    x_flat = x.reshape(BS, D)
    wts8, idx8 = _router(x_flat, Wr, br)
    flat_i = idx8[:, :K]
    flat_w = wts8[:, :K]

    e_flat = flat_i.reshape(A)
    oh = (e_flat[:, None] == jnp.arange(E, dtype=e_flat.dtype)).astype(jnp.int32)
    counts = jnp.sum(oh, axis=0)
    rank = jnp.take_along_axis(
        jnp.cumsum(oh, axis=0), e_flat[:, None], axis=1)[:, 0] - 1
    c_pad = ((counts + _TM - 1) // _TM) * _TM
    starts = jnp.concatenate(
        [jnp.zeros((1,), jnp.int32), jnp.cumsum(c_pad)[:-1].astype(jnp.int32)])
    dest = starts[e_flat] + rank
    src_rows = jnp.zeros((P,), jnp.int32).at[dest].set(
        jnp.arange(A, dtype=jnp.int32) // K)
    w_sorted = jnp.zeros((P,), jnp.float32).at[dest].set(flat_w.reshape(A))
    ntiles = P // _TM
    offs = jnp.arange(ntiles, dtype=jnp.int32) * _TM
    gids = (jnp.searchsorted(starts, offs, side="right") - 1).astype(jnp.int32)

    xs = _make_sc_gather(BS, D, P, 32, jnp.float32)(x_flat, src_rows)

    h = _expert_mm(xs, We, be, w_sorted.reshape(ntiles, 1, _TM), gids)

    h_w = lax.bitcast_convert_type(
        h.reshape(P, D // 2, 2), jnp.int32)
    pos_ab = jnp.concatenate([dest[0::2], dest[1::2]])
    gab_w = _make_sc_gather(P, D // 2, A, 32, jnp.int32)(h_w, pos_ab)
    gab = lax.bitcast_convert_type(gab_w, jnp.bfloat16).reshape(A, D)

    out = _out_proj(gab, x_flat, Wo, bo, g)
    return out.reshape(B, S, D)


# speedup vs baseline: 2.2735x; 2.2735x over previous
import functools

import jax
import jax.numpy as jnp
from jax import lax
from jax.experimental import pallas as pl
from jax.experimental.pallas import tpu as pltpu
from jax.experimental.pallas import tpu_sc as plsc

_NC = 2
_NS = 16
_NW = _NC * _NS

_TM = 256


def _router_body(x_ref, wr_ref, br_ref, w_ref, i_ref):
    xb = x_ref[...]
    logits = lax.dot_general(
        xb, wr_ref[...], (((1,), (1,)), ((), ())),
        preferred_element_type=jnp.float32,
    ) + br_ref[...]
    e_iota = lax.broadcasted_iota(jnp.int32, logits.shape, 1)
    m1 = jnp.max(logits, axis=1, keepdims=True)
    i1 = jnp.argmax(logits, axis=1).astype(jnp.int32)
    oh1 = e_iota == i1[:, None]
    l2 = jnp.where(oh1, -jnp.inf, logits)
    m2 = jnp.max(l2, axis=1, keepdims=True)
    i2 = jnp.argmax(l2, axis=1).astype(jnp.int32)
    b = jnp.exp(m2 - m1)
    w1 = 1.0 / (1.0 + b)
    w2 = 1.0 - w1
    w_ref[...] = jnp.where(e_iota == 0, w1, jnp.where(e_iota == 1, w2, 0.0))
    i_ref[...] = jnp.where(e_iota == 0, i1[:, None],
                           jnp.where(e_iota == 1, i2[:, None], 0))


def _router(x_flat, Wr, br):
    BS, D = x_flat.shape
    E = Wr.shape[0]
    TMA = 512
    grid = (BS // TMA,)
    return pl.pallas_call(
        _router_body,
        grid=grid,
        in_specs=[
            pl.BlockSpec((TMA, D), lambda i: (i, 0)),
            pl.BlockSpec((E, D), lambda i: (0, 0)),
            pl.BlockSpec((1, E), lambda i: (0, 0)),
        ],
        out_specs=[
            pl.BlockSpec((TMA, E), lambda i: (i, 0)),
            pl.BlockSpec((TMA, E), lambda i: (i, 0)),
        ],
        out_shape=[
            jax.ShapeDtypeStruct((BS, E), jnp.float32),
            jax.ShapeDtypeStruct((BS, E), jnp.int32),
        ],
    )(x_flat, Wr, br.reshape(1, E))


def _make_sc_gather(V, D, N, CH, dtype):
    assert N % _NW == 0
    rows_pw = N // _NW
    assert rows_pw % CH == 0 and rows_pw % 8 == 0
    n_ch = rows_pw // CH
    assert n_ch % 2 == 0 and n_ch >= 4
    mesh = plsc.VectorSubcoreMesh(
        core_axis_name="c", subcore_axis_name="s",
        num_cores=_NC, num_subcores=_NS,
    )

    @functools.partial(
        pl.kernel,
        out_type=jax.ShapeDtypeStruct((N, D), dtype),
        mesh=mesh,
        scratch_types=[
            pltpu.VMEM((rows_pw,), jnp.int32),
            pltpu.VMEM((2, CH, D), dtype),
            pltpu.SemaphoreType.DMA,
            pltpu.SemaphoreType.DMA,
            pltpu.SemaphoreType.DMA,
            pltpu.SemaphoreType.DMA,
        ],
    )
    def gather(table_hbm, idx_hbm, out_hbm, idx_v, rows_v, g0, g1, w0, w1):
        wid = lax.axis_index("s") * _NC + lax.axis_index("c")
        base = wid * rows_pw
        g_sems = (g0, g1)
        w_sems = (w0, w1)
        pltpu.sync_copy(idx_hbm.at[pl.ds(base, rows_pw)], idx_v)

        def fire_gather(j, b):
            pltpu.async_copy(
                table_hbm.at[idx_v.at[pl.ds(j * CH, CH)]], rows_v.at[b],
                g_sems[b])

        fire_gather(0, 0)
        fire_gather(1, 1)

        @pl.loop(0, n_ch, step=2)
        def _pair(i):
            for b in range(2):
                j = i + b
                pltpu.make_async_copy(
                    table_hbm.at[idx_v.at[pl.ds(0, CH)]], rows_v.at[b],
                    g_sems[b]).wait()
                pltpu.async_copy(
                    rows_v.at[b], out_hbm.at[pl.ds(base + j * CH, CH)],
                    w_sems[b])

                @pl.when(j + 2 < n_ch)
                def _refill():
                    pltpu.make_async_copy(
                        rows_v.at[b], out_hbm.at[pl.ds(base, CH)],
                        w_sems[b]).wait()
                    fire_gather(j + 2, b)

        for b in range(2):
            pltpu.make_async_copy(
                rows_v.at[b], out_hbm.at[pl.ds(base, CH)], w_sems[b]).wait()

    return gather


def _expert_body(gid_ref, xs_ref, we_ref, be_ref, ws_ref, out_ref):
    g = gid_ref[pl.program_id(0)]
    xb = xs_ref[...]
    wg = we_ref[g]
    z = lax.dot_general(
        xb, wg, (((1,), (1,)), ((), ())),
        preferred_element_type=jnp.float32,
    ) + be_ref[g][None, :]
    h = z * (1.0 / (1.0 + jnp.exp(-z)))
    w = ws_ref[0, 0][:, None]
    out_ref[...] = h * w


def _expert_mm(xs, We, be, w_tiles, gids):
    P, D = xs.shape
    E = We.shape[0]
    ntiles = P // _TM
    return pl.pallas_call(
        _expert_body,
        grid=(ntiles,),
        in_specs=[
            pl.BlockSpec(memory_space=pltpu.SMEM),
            pl.BlockSpec((_TM, D), lambda i: (i, 0)),
            pl.BlockSpec((E, D, D), lambda i: (0, 0, 0)),
            pl.BlockSpec((E, D), lambda i: (0, 0)),
            pl.BlockSpec((1, 1, _TM), lambda i: (i, 0, 0)),
        ],
        out_specs=pl.BlockSpec((_TM, D), lambda i: (i, 0)),
        out_shape=jax.ShapeDtypeStruct((P, D), jnp.float32),
        compiler_params=pltpu.CompilerParams(
            dimension_semantics=("arbitrary",),
        ),
    )(gids, xs, We, be, w_tiles)


def _out_body(ga_ref, gb_ref, x_ref, wo_ref, bo_ref, g_ref, o_ref):
    c = ga_ref[...] + gb_ref[...]
    z = lax.dot_general(
        c, wo_ref[...], (((1,), (1,)), ((), ())),
        preferred_element_type=jnp.float32,
    ) + bo_ref[...]
    y = x_ref[...] + z
    ms = jnp.mean(y * y, axis=1, keepdims=True)
    o_ref[...] = g_ref[...] * (y * lax.rsqrt(ms + 1e-6))


def _out_proj(gab, x_flat, Wo, bo, g):
    BS, D = x_flat.shape
    TMD = 256
    nb = BS // TMD
    return pl.pallas_call(
        _out_body,
        grid=(nb,),
        in_specs=[
            pl.BlockSpec((TMD, D), lambda i: (i, 0)),
            pl.BlockSpec((TMD, D), lambda i, nb=nb: (i + nb, 0)),
            pl.BlockSpec((TMD, D), lambda i: (i, 0)),
            pl.BlockSpec((D, D), lambda i: (0, 0)),
            pl.BlockSpec((1, D), lambda i: (0, 0)),
            pl.BlockSpec((1, D), lambda i: (0, 0)),
        ],
        out_specs=pl.BlockSpec((TMD, D), lambda i: (i, 0)),
        out_shape=jax.ShapeDtypeStruct((BS, D), jnp.float32),
    )(gab, gab, x_flat, Wo, bo.reshape(1, D), g.reshape(1, D))


def kernel(x, Wr, br, We, be, Wo, bo, g):
    B, S, D = x.shape
    E = Wr.shape[0]
    K = 2
    BS = B * S
    A = BS * K
    P = A + E * _TM

    x_flat = x.reshape(BS, D)
    wts8, idx8 = _router(x_flat, Wr, br)
    flat_i = idx8[:, :K]
    flat_w = wts8[:, :K]

    e_flat = flat_i.reshape(A)
    oh = (e_flat[:, None] == jnp.arange(E, dtype=e_flat.dtype)).astype(jnp.int32)
    counts = jnp.sum(oh, axis=0)
    rank = jnp.take_along_axis(
        jnp.cumsum(oh, axis=0), e_flat[:, None], axis=1)[:, 0] - 1
    c_pad = ((counts + _TM - 1) // _TM) * _TM
    starts = jnp.concatenate(
        [jnp.zeros((1,), jnp.int32), jnp.cumsum(c_pad)[:-1].astype(jnp.int32)])
    dest = starts[e_flat] + rank
    src_rows = jnp.zeros((P,), jnp.int32).at[dest].set(
        jnp.arange(A, dtype=jnp.int32) // K)
    w_sorted = jnp.zeros((P,), jnp.float32).at[dest].set(flat_w.reshape(A))
    ntiles = P // _TM
    offs = jnp.arange(ntiles, dtype=jnp.int32) * _TM
    gids = (jnp.searchsorted(starts, offs, side="right") - 1).astype(jnp.int32)

    xs = _make_sc_gather(BS, D, P, 32, jnp.float32)(x_flat, src_rows)

    h = _expert_mm(xs, We, be, w_sorted.reshape(ntiles, 1, _TM), gids)

    pos_ab = jnp.concatenate([dest[0::2], dest[1::2]])
    gab = _make_sc_gather(P, D, A, 32, jnp.float32)(h, pos_ab)

    out = _out_proj(gab, x_flat, Wo, bo, g)
    return out.reshape(B, S, D)
